# baseline (speedup 1.0000x reference)
import functools

import jax
import jax.numpy as jnp
from jax import lax
from jax.experimental import pallas as pl
from jax.experimental.pallas import tpu as pltpu
from jax.experimental.pallas import tpu_sc as plsc

B = 16384
HIDDEN = 256
EMBED = 128
WPR = EMBED // 2

SEG_OFF = (0, 288, 560)
F_ROWS = 832

NC = 2
NS = 16
LANES = 16
NW = NC * NS
BPW = B // NW
OCH = 256
NCHUNK = BPW // OCH


def _pack_words(p):
    ph = p.astype(jnp.bfloat16)
    lo = lax.convert_element_type(
        lax.bitcast_convert_type(ph[:, 0:WPR], jnp.uint16), jnp.uint32)
    hi = lax.convert_element_type(
        lax.bitcast_convert_type(ph[:, WPR:EMBED], jnp.uint16), jnp.uint32)
    return lax.bitcast_convert_type(lo | (hi << 16), jnp.int32)


def _fuse_tables_kernel(dep_ref, sid_ref, eid_ref, w1_ref, b1_ref, f_ref):
    f_ref[0:288, :] = _pack_words(
        jnp.dot(dep_ref[:], w1_ref[0:256, :],
                preferred_element_type=jnp.float32) + b1_ref[:])
    f_ref[288:545, :] = _pack_words(jnp.dot(
        sid_ref[:], w1_ref[256:512, :], preferred_element_type=jnp.float32))
    f_ref[560:817, :] = _pack_words(jnp.dot(
        eid_ref[:], w1_ref[512:768, :], preferred_element_type=jnp.float32))


def _sc_embed_sum(f_hbm, idx_hbm, out_hbm,
                  f_v, f_sh, idx0, idx1, idx2, oa, ob,
                  fsem, osem_a, osem_b):
    sid = lax.axis_index("s")
    wid = sid * NC + lax.axis_index("c")
    base = wid * BPW
    @pl.when(sid == 0)
    def _():
        pltpu.sync_copy(f_hbm, f_sh)
    pltpu.sync_copy(idx_hbm.at[pl.ds(0 * B + base, BPW)], idx0)
    pltpu.sync_copy(idx_hbm.at[pl.ds(1 * B + base, BPW)], idx1)
    pltpu.sync_copy(idx_hbm.at[pl.ds(2 * B + base, BPW)], idx2)
    plsc.subcore_barrier()
    pltpu.sync_copy(f_sh, f_v)

    obufs = ((oa, osem_a), (ob, osem_b))
    out_cps = [None, None]
    for c in range(NCHUNK):
        obuf, osem = obufs[c % 2]
        if out_cps[c % 2] is not None:
            out_cps[c % 2].wait()

        def group_body(g, _):
            s = c * OCH + g * LANES
            lanev = lax.iota(jnp.int32, LANES)
            kv0 = idx0[pl.ds(s, LANES)]
            kv1 = idx1[pl.ds(s, LANES)]
            kv2 = idx2[pl.ds(s, LANES)]
            rowv = lanev * EMBED + g * (LANES * EMBED)

            @plsc.parallel_loop(0, WPR, step=1, unroll=4)
            def col_body(w, ):
                cvec = (w + lanev) & (WPR - 1)
                v = (plsc.bitcast(plsc.load_gather(f_v, [kv0 + cvec]),
                                  jnp.bfloat16)
                     + plsc.bitcast(plsc.load_gather(f_v, [kv1 + cvec]),
                                    jnp.bfloat16)
                     + plsc.bitcast(plsc.load_gather(f_v, [kv2 + cvec]),
                                    jnp.bfloat16))
                v = jnp.maximum(v, jnp.bfloat16(0.0))
                lo, hi = plsc.unpack(v, format=plsc.PackFormat.INTERLEAVED,
                                     preferred_element_type=jnp.float32)
                addr = rowv + cvec
                plsc.store_scatter(obuf, [addr], lo)
                plsc.store_scatter(obuf, [addr + WPR], hi)

            return 0

        lax.fori_loop(0, OCH // LANES, group_body, 0)
        out_cps[c % 2] = pltpu.async_copy(
            obuf, out_hbm.at[pl.ds((base + c * OCH) * EMBED, OCH * EMBED)],
            osem)
    for cp in out_cps:
        if cp is not None:
            cp.wait()


def _mlp2_kernel(x_ref, w2_ref, b2_ref, o_ref):
    o_ref[:] = (
        jnp.dot(x_ref[:], w2_ref[:], preferred_element_type=jnp.float32)
        + b2_ref[:]
    )


def kernel(attr, dep_table, sid_table, eid_table, W1, b1, W2, b2):
    f32 = jnp.float32

    F = pl.pallas_call(
        _fuse_tables_kernel,
        out_shape=jax.ShapeDtypeStruct((F_ROWS, WPR), jnp.int32),
    )(dep_table, sid_table, eid_table, W1, b1.reshape(1, EMBED))
    f_words = F.reshape(F_ROWS * WPR)

    off = jnp.array(SEG_OFF, jnp.int32)
    idx = ((attr + off[None, :]) * WPR).T.reshape(3 * B)

    mesh = plsc.VectorSubcoreMesh(core_axis_name="c", subcore_axis_name="s")
    sc_call = functools.partial(
        pl.kernel,
        mesh=mesh,
        compiler_params=pltpu.CompilerParams(needs_layout_passes=False),
        out_type=jax.ShapeDtypeStruct((B * EMBED,), f32),
        scratch_types=[
            pltpu.VMEM((F_ROWS * WPR,), jnp.int32),
            pltpu.VMEM_SHARED((F_ROWS * WPR,), jnp.int32),
            pltpu.VMEM((BPW,), jnp.int32),
            pltpu.VMEM((BPW,), jnp.int32),
            pltpu.VMEM((BPW,), jnp.int32),
            pltpu.VMEM((OCH * EMBED,), f32),
            pltpu.VMEM((OCH * EMBED,), f32),
            pltpu.SemaphoreType.DMA,
            pltpu.SemaphoreType.DMA,
            pltpu.SemaphoreType.DMA,
        ],
    )(_sc_embed_sum)
    pre = sc_call(f_words, idx).reshape(B, EMBED)

    BM = 8192
    out = pl.pallas_call(
        _mlp2_kernel,
        grid=(B // BM,),
        in_specs=[
            pl.BlockSpec((BM, EMBED), lambda i: (i, 0)),
            pl.BlockSpec((EMBED, EMBED), lambda i: (0, 0)),
            pl.BlockSpec((1, EMBED), lambda i: (0, 0)),
        ],
        out_specs=pl.BlockSpec((BM, EMBED), lambda i: (i, 0)),
        out_shape=jax.ShapeDtypeStruct((B, EMBED), f32),
    )(pre, W2, b2.reshape(1, EMBED))
    return out


# speedup vs baseline: 1.0108x; 1.0108x over previous
import functools

import jax
import jax.numpy as jnp
from jax import lax
from jax.experimental import pallas as pl
from jax.experimental.pallas import tpu as pltpu
from jax.experimental.pallas import tpu_sc as plsc

B = 16384
HIDDEN = 256
EMBED = 128
WPR = EMBED // 2

SEG_OFF = (0, 288, 560)
F_ROWS = 832

NC = 2
NS = 16
LANES = 16
NW = NC * NS
BPW = B // NW
OCH = 128
NCHUNK = BPW // OCH


def _pack_words(p):
    ph = p.astype(jnp.bfloat16)
    lo = lax.convert_element_type(
        lax.bitcast_convert_type(ph[:, 0:WPR], jnp.uint16), jnp.uint32)
    hi = lax.convert_element_type(
        lax.bitcast_convert_type(ph[:, WPR:EMBED], jnp.uint16), jnp.uint32)
    return lax.bitcast_convert_type(lo | (hi << 16), jnp.int32)


def _fuse_tables_kernel(dep_ref, sid_ref, eid_ref, w1_ref, b1_ref, f_ref):
    f_ref[0:288, :] = _pack_words(
        jnp.dot(dep_ref[:], w1_ref[0:256, :],
                preferred_element_type=jnp.float32) + b1_ref[:])
    f_ref[288:545, :] = _pack_words(jnp.dot(
        sid_ref[:], w1_ref[256:512, :], preferred_element_type=jnp.float32))
    f_ref[560:817, :] = _pack_words(jnp.dot(
        eid_ref[:], w1_ref[512:768, :], preferred_element_type=jnp.float32))


def _sc_embed_sum(f_hbm, idx_hbm, out_hbm,
                  f_v, f_sh, idx0, idx1, idx2, oa, ob,
                  fsem, osem_a, osem_b):
    sid = lax.axis_index("s")
    wid = sid * NC + lax.axis_index("c")
    base = wid * BPW
    @pl.when(sid == 0)
    def _():
        pltpu.sync_copy(f_hbm, f_sh)
    pltpu.sync_copy(idx_hbm.at[pl.ds(0 * B + base, BPW)], idx0)
    pltpu.sync_copy(idx_hbm.at[pl.ds(1 * B + base, BPW)], idx1)
    pltpu.sync_copy(idx_hbm.at[pl.ds(2 * B + base, BPW)], idx2)
    plsc.subcore_barrier()
    pltpu.sync_copy(f_sh, f_v)

    obufs = ((oa, osem_a), (ob, osem_b))
    out_cps = [None, None]
    for c in range(NCHUNK):
        obuf, osem = obufs[c % 2]
        if out_cps[c % 2] is not None:
            out_cps[c % 2].wait()

        def group_body(g, _):
            s = c * OCH + g * LANES
            lanev = lax.iota(jnp.int32, LANES)
            kv0 = idx0[pl.ds(s, LANES)]
            kv1 = idx1[pl.ds(s, LANES)]
            kv2 = idx2[pl.ds(s, LANES)]
            rowv = lanev * EMBED + g * (LANES * EMBED)

            @plsc.parallel_loop(0, WPR, step=1, unroll=4)
            def col_body(w, ):
                cvec = (w + lanev) & (WPR - 1)
                v = (plsc.bitcast(plsc.load_gather(f_v, [kv0 + cvec]),
                                  jnp.bfloat16)
                     + plsc.bitcast(plsc.load_gather(f_v, [kv1 + cvec]),
                                    jnp.bfloat16)
                     + plsc.bitcast(plsc.load_gather(f_v, [kv2 + cvec]),
                                    jnp.bfloat16))
                v = jnp.maximum(v, jnp.bfloat16(0.0))
                lo, hi = plsc.unpack(v, format=plsc.PackFormat.INTERLEAVED,
                                     preferred_element_type=jnp.float32)
                addr = rowv + cvec
                plsc.store_scatter(obuf, [addr], lo)
                plsc.store_scatter(obuf, [addr + WPR], hi)

            return 0

        lax.fori_loop(0, OCH // LANES, group_body, 0)
        out_cps[c % 2] = pltpu.async_copy(
            obuf, out_hbm.at[pl.ds((base + c * OCH) * EMBED, OCH * EMBED)],
            osem)
    for cp in out_cps:
        if cp is not None:
            cp.wait()


def _mlp2_kernel(x_ref, w2_ref, b2_ref, o_ref):
    o_ref[:] = (
        jnp.dot(x_ref[:], w2_ref[:], preferred_element_type=jnp.float32)
        + b2_ref[:]
    )


def kernel(attr, dep_table, sid_table, eid_table, W1, b1, W2, b2):
    f32 = jnp.float32

    F = pl.pallas_call(
        _fuse_tables_kernel,
        out_shape=jax.ShapeDtypeStruct((F_ROWS, WPR), jnp.int32),
    )(dep_table, sid_table, eid_table, W1, b1.reshape(1, EMBED))
    f_words = F.reshape(F_ROWS * WPR)

    off = jnp.array(SEG_OFF, jnp.int32)
    idx = ((attr + off[None, :]) * WPR).T.reshape(3 * B)

    mesh = plsc.VectorSubcoreMesh(core_axis_name="c", subcore_axis_name="s")
    sc_call = functools.partial(
        pl.kernel,
        mesh=mesh,
        compiler_params=pltpu.CompilerParams(needs_layout_passes=False),
        out_type=jax.ShapeDtypeStruct((B * EMBED,), f32),
        scratch_types=[
            pltpu.VMEM((F_ROWS * WPR,), jnp.int32),
            pltpu.VMEM_SHARED((F_ROWS * WPR,), jnp.int32),
            pltpu.VMEM((BPW,), jnp.int32),
            pltpu.VMEM((BPW,), jnp.int32),
            pltpu.VMEM((BPW,), jnp.int32),
            pltpu.VMEM((OCH * EMBED,), f32),
            pltpu.VMEM((OCH * EMBED,), f32),
            pltpu.SemaphoreType.DMA,
            pltpu.SemaphoreType.DMA,
            pltpu.SemaphoreType.DMA,
        ],
    )(_sc_embed_sum)
    pre = sc_call(f_words, idx).reshape(B, EMBED)

    BM = 8192
    out = pl.pallas_call(
        _mlp2_kernel,
        grid=(B // BM,),
        in_specs=[
            pl.BlockSpec((BM, EMBED), lambda i: (i, 0)),
            pl.BlockSpec((EMBED, EMBED), lambda i: (0, 0)),
            pl.BlockSpec((1, EMBED), lambda i: (0, 0)),
        ],
        out_specs=pl.BlockSpec((BM, EMBED), lambda i: (i, 0)),
        out_shape=jax.ShapeDtypeStruct((B, EMBED), f32),
    )(pre, W2, b2.reshape(1, EMBED))
    return out
